# Initial kernel scaffold; baseline (speedup 1.0000x reference)
import functools

import jax
import jax.numpy as jnp
from jax import lax
from jax.experimental import pallas as pl
from jax.experimental.pallas import tpu as pltpu
from jax.experimental.pallas import tpu_sc as plsc

_BATCH = 16384
_FIELDS = 26
_TOTAL = _BATCH * _FIELDS
_EMB = 16
_OUT = 64

_NW = 32
_B_PER_W = _TOTAL // _NW
_CHUNK = 1664
_N_CHUNKS = _B_PER_W // _CHUNK


def _make_gather():
    mesh = plsc.VectorSubcoreMesh(core_axis_name="c", subcore_axis_name="s")

    @functools.partial(
        pl.kernel,
        mesh=mesh,
        out_type=jax.ShapeDtypeStruct((_TOTAL, _EMB), jnp.float32),
        scratch_types=[
            pltpu.VMEM((_B_PER_W,), jnp.int32),
            pltpu.VMEM((_CHUNK, _EMB), jnp.float32),
            pltpu.SemaphoreType.DMA,
        ],
    )
    def gather_k(idx_hbm, table_hbm, out_hbm, idx_v, rows_v, sem):
        wid = lax.axis_index("s") * 2 + lax.axis_index("c")
        base = wid * _B_PER_W
        pltpu.sync_copy(idx_hbm.at[pl.ds(base, _B_PER_W)], idx_v)

        def body(i, carry):
            off = i * _CHUNK
            pltpu.async_copy(
                table_hbm.at[idx_v.at[pl.ds(off, _CHUNK)]], rows_v, sem
            ).wait()
            pltpu.sync_copy(rows_v, out_hbm.at[pl.ds(base + off, _CHUNK)])
            return carry

        lax.fori_loop(0, _N_CHUNKS, body, 0)

    return gather_k


_gather = _make_gather()

_BLK = 4096


def _mm_body(emb_ref, wt_ref, b_ref, out_ref):
    out_ref[...] = (
        jnp.dot(emb_ref[...], wt_ref[...], preferred_element_type=jnp.float32)
        + b_ref[...]
    )


def _project(emb, wt, b2d):
    return pl.pallas_call(
        _mm_body,
        grid=(_TOTAL // _BLK,),
        in_specs=[
            pl.BlockSpec((_BLK, _EMB), lambda i: (i, 0)),
            pl.BlockSpec((_EMB, _OUT), lambda i: (0, 0)),
            pl.BlockSpec((1, _OUT), lambda i: (0, 0)),
        ],
        out_specs=pl.BlockSpec((_BLK, _OUT), lambda i: (i, 0)),
        out_shape=jax.ShapeDtypeStruct((_TOTAL, _OUT), jnp.float32),
    )(emb, wt, b2d)


def kernel(x, table, W, b):
    idx = x.reshape(_TOTAL).astype(jnp.int32)
    emb = _gather(idx, table)
    out = _project(emb, W.T, b.reshape(1, _OUT))
    return out.reshape(_BATCH, _FIELDS, _OUT)


# speedup vs baseline: 8.3281x; 8.3281x over previous
import functools

import jax
import jax.numpy as jnp
from jax import lax
from jax.experimental import pallas as pl
from jax.experimental.pallas import tpu as pltpu
from jax.experimental.pallas import tpu_sc as plsc

_BATCH = 16384
_FIELDS = 26
_TOTAL = _BATCH * _FIELDS
_EMB = 16
_OUT = 64

_NW = 32
_B_PER_W = _TOTAL // _NW
_CHUNK = 1664
_N_CHUNKS = _B_PER_W // _CHUNK


def _make_gather():
    mesh = plsc.VectorSubcoreMesh(core_axis_name="c", subcore_axis_name="s")

    @functools.partial(
        pl.kernel,
        mesh=mesh,
        compiler_params=pltpu.CompilerParams(use_tc_tiling_on_sc=False),
        out_type=jax.ShapeDtypeStruct((_TOTAL, _EMB), jnp.float32),
        scratch_types=[
            pltpu.VMEM((_B_PER_W,), jnp.int32),
            pltpu.VMEM((_CHUNK, _EMB), jnp.float32),
            pltpu.SemaphoreType.DMA,
        ],
    )
    def gather_k(idx_hbm, table_hbm, out_hbm, idx_v, rows_v, sem):
        wid = lax.axis_index("s") * 2 + lax.axis_index("c")
        base = wid * _B_PER_W
        pltpu.sync_copy(idx_hbm.at[pl.ds(base, _B_PER_W)], idx_v)

        def body(i, carry):
            off = i * _CHUNK
            pltpu.async_copy(
                table_hbm.at[idx_v.at[pl.ds(off, _CHUNK)]], rows_v, sem
            ).wait()
            pltpu.sync_copy(rows_v, out_hbm.at[pl.ds(base + off, _CHUNK)])
            return carry

        lax.fori_loop(0, _N_CHUNKS, body, 0)

    return gather_k


_gather = _make_gather()

_BLK = 4096


def _mm_body(emb_ref, wt_ref, b_ref, out_ref):
    out_ref[...] = (
        jnp.dot(emb_ref[...], wt_ref[...], preferred_element_type=jnp.float32)
        + b_ref[...]
    )


def _project(emb, wt, b2d):
    return pl.pallas_call(
        _mm_body,
        grid=(_TOTAL // _BLK,),
        in_specs=[
            pl.BlockSpec((_BLK, _EMB), lambda i: (i, 0)),
            pl.BlockSpec((_EMB, _OUT), lambda i: (0, 0)),
            pl.BlockSpec((1, _OUT), lambda i: (0, 0)),
        ],
        out_specs=pl.BlockSpec((_BLK, _OUT), lambda i: (i, 0)),
        out_shape=jax.ShapeDtypeStruct((_TOTAL, _OUT), jnp.float32),
    )(emb, wt, b2d)


def kernel(x, table, W, b):
    idx = x.reshape(_TOTAL).astype(jnp.int32)
    emb = _gather(idx, table)
    out = _project(emb, W.T, b.reshape(1, _OUT))
    return out.reshape(_BATCH, _FIELDS, _OUT)


# speedup vs baseline: 49.4592x; 5.9388x over previous
import functools

import jax
import jax.numpy as jnp
from jax import lax
from jax.experimental import pallas as pl
from jax.experimental.pallas import tpu as pltpu
from jax.experimental.pallas import tpu_sc as plsc

_BATCH = 16384
_FIELDS = 26
_TOTAL = _BATCH * _FIELDS
_EMB = 16
_OUT = 64
_NUM_EMB = 1000000

_NW = 32
_B_PER_W = _TOTAL // _NW
_B_PER_T = _TOTAL // 16


def _make_mega():
    mesh = plsc.VectorSubcoreMesh(core_axis_name="c", subcore_axis_name="s")

    @functools.partial(
        pl.kernel,
        mesh=mesh,
        compiler_params=pltpu.CompilerParams(
            use_tc_tiling_on_sc=True, needs_layout_passes=False
        ),
        out_type=jax.ShapeDtypeStruct((_EMB, _TOTAL), jnp.float32),
        scratch_types=[
            pltpu.VMEM((_B_PER_T,), jnp.int32),
            pltpu.VMEM((_B_PER_T,), jnp.float32),
            pltpu.VMEM_SHARED((_NUM_EMB,), jnp.float32),
            pltpu.SemaphoreType.DMA,
        ],
    )
    def mega_k(idx_hbm, tT_hbm, emb_hbm, idx_v, buf_v, plane, sem):
        sid = lax.axis_index("s")
        core = lax.axis_index("c")
        e0 = core * (_EMB // 2)
        base = sid * _B_PER_T

        @pl.when(sid == 0)
        def _stage0():
            pltpu.async_copy(tT_hbm.at[e0], plane, sem)

        pltpu.sync_copy(idx_hbm.at[pl.ds(base, _B_PER_T)], idx_v)

        @pl.when(sid == 0)
        def _wait0():
            pltpu.make_async_copy(tT_hbm.at[e0], plane, sem).wait()

        plsc.subcore_barrier()
        for e_local in range(_EMB // 2):
            e = e0 + e_local
            pltpu.sync_copy(plane.at[idx_v], buf_v)
            pltpu.sync_copy(buf_v, emb_hbm.at[e, pl.ds(base, _B_PER_T)])
            plsc.subcore_barrier()
            if e_local + 1 < _EMB // 2:

                @pl.when(sid == 0)
                def _stage_next():
                    pltpu.sync_copy(tT_hbm.at[e + 1], plane)

                plsc.subcore_barrier()

    return mega_k


_mega = _make_mega()


def _mm_body(wt_ref, b_ref, emb_ref, out_ref):
    res = jax.lax.dot_general(
        wt_ref[...],
        emb_ref[...],
        dimension_numbers=(((0,), (0,)), ((), ())),
        preferred_element_type=jnp.float32,
    )
    out_ref[0, :, :] = res + b_ref[...].reshape(_OUT, 1)


def _project(wt, b, emb):
    return pl.pallas_call(
        _mm_body,
        grid=(_FIELDS,),
        in_specs=[
            pl.BlockSpec((_EMB, _OUT), lambda f: (0, 0)),
            pl.BlockSpec((_OUT,), lambda f: (0,)),
            pl.BlockSpec((_EMB, _BATCH), lambda f: (0, f)),
        ],
        out_specs=pl.BlockSpec((1, _OUT, _BATCH), lambda f: (f, 0, 0)),
        out_shape=jax.ShapeDtypeStruct((_FIELDS, _OUT, _BATCH), jnp.float32),
    )(wt, b, emb)


def kernel(x, table, W, b):
    idx = x.T.reshape(_TOTAL).astype(jnp.int32)
    emb = _mega(idx, table.T)
    outp = _project(W.T, b, emb)
    return jnp.transpose(outp, (2, 0, 1))
